# baseline (speedup 1.0000x reference)
import jax
import jax.numpy as jnp
import numpy as np
from jax import lax
from jax.experimental import pallas as pl
from jax.experimental.pallas import tpu as pltpu
from jax.experimental.pallas import tpu_sc as plsc

H, D = 8, 16
N = 10000
E = 320000
DIN, DH, DOUT = 128, 128, 64

NTILES = 16
CHUNK = 80
SUPER = 8
EPT = 20480
EPAD = EPT * NTILES
NCHUNKS = EPT // CHUNK
NSUPER = NCHUNKS // SUPER
ROWS_PT = 640
NPAD = ROWS_PT * NTILES
W_ACC = 136
BN = 1000
GRID = N // BN


def _k1_body(xo_ref, xu_ref, wot_ref, wut_ref, bo_ref, bu_ref, p_ref,
             a_suo_ref, a_duo_ref, a_soo_ref, a_doo_ref,
             hob_ref, hub_ref, tsuo_ref, tduo_ref, tsoo_ref, tdoo_ref):
    ho = jnp.dot(xo_ref[...], wot_ref[...],
                 preferred_element_type=jnp.float32) + bo_ref[...]
    hu = jnp.dot(xu_ref[...], wut_ref[...],
                 preferred_element_type=jnp.float32) + bu_ref[...]
    hob_ref[...] = jnp.dot(ho, p_ref[...],
                           preferred_element_type=jnp.float32).astype(jnp.bfloat16)
    hub_ref[...] = jnp.dot(hu, p_ref[...],
                           preferred_element_type=jnp.float32).astype(jnp.bfloat16)
    tsuo_ref[...] = jnp.dot(hu, a_suo_ref[...], preferred_element_type=jnp.float32)
    tduo_ref[...] = jnp.dot(ho, a_duo_ref[...], preferred_element_type=jnp.float32)
    tsoo_ref[...] = jnp.dot(ho, a_soo_ref[...], preferred_element_type=jnp.float32)
    tdoo_ref[...] = jnp.dot(ho, a_doo_ref[...], preferred_element_type=jnp.float32)


def _blockdiag(att):
    a = jnp.einsum('hd,hk->hdk', att[0], jnp.eye(H, dtype=att.dtype))
    return jnp.pad(a.reshape(H * D, H), ((0, 0), (0, 8)))


def _interleave_perm():
    p = np.zeros((DH, DH), np.float32)
    for j in range(4):
        for t in range(16):
            p[32 * j + t, 32 * j + 2 * t] = 1.0
            p[32 * j + 16 + t, 32 * j + 2 * t + 1] = 1.0
    return jnp.asarray(p)


def _stage1(x_order, x_user, W_order, b_order, W_user, b_user,
            att_src_uo, att_dst_uo, att_src_oo, att_dst_oo):
    row = lambda i: (i, 0)
    fixed = lambda i: (0, 0)
    f32 = jnp.float32
    return pl.pallas_call(
        _k1_body,
        grid=(GRID,),
        in_specs=[
            pl.BlockSpec((BN, DIN), row), pl.BlockSpec((BN, DIN), row),
            pl.BlockSpec((DIN, DH), fixed), pl.BlockSpec((DIN, DH), fixed),
            pl.BlockSpec((1, DH), fixed), pl.BlockSpec((1, DH), fixed),
            pl.BlockSpec((DH, DH), fixed),
            pl.BlockSpec((DH, 16), fixed), pl.BlockSpec((DH, 16), fixed),
            pl.BlockSpec((DH, 16), fixed), pl.BlockSpec((DH, 16), fixed),
        ],
        out_specs=[
            pl.BlockSpec((BN, DH), row), pl.BlockSpec((BN, DH), row),
            pl.BlockSpec((BN, 16), row), pl.BlockSpec((BN, 16), row),
            pl.BlockSpec((BN, 16), row), pl.BlockSpec((BN, 16), row),
        ],
        out_shape=[
            jax.ShapeDtypeStruct((N, DH), jnp.bfloat16),
            jax.ShapeDtypeStruct((N, DH), jnp.bfloat16),
            jax.ShapeDtypeStruct((N, 16), f32), jax.ShapeDtypeStruct((N, 16), f32),
            jax.ShapeDtypeStruct((N, 16), f32), jax.ShapeDtypeStruct((N, 16), f32),
        ],
    )(x_order, x_user, W_order.T, W_user.T,
      b_order.reshape(1, DH), b_user.reshape(1, DH), _interleave_perm(),
      _blockdiag(att_src_uo), _blockdiag(att_dst_uo),
      _blockdiag(att_src_oo), _blockdiag(att_dst_oo))


def _lane_bcast(x, h):
    idx = jnp.full((16, 1), h, jnp.int32)
    dn = lax.GatherDimensionNumbers(
        offset_dims=(), collapsed_slice_dims=(0,), start_index_map=(0,))
    return lax.gather(x, idx, dn, (1,),
                      mode=lax.GatherScatterMode.PROMISE_IN_BOUNDS)


def _sc_body(src_uo, dst_uo, src_oo, dst_oo,
             tsuo, tduo, hub, tsoo, tdoo, hob,
             out_hbm,
             acc, src_sb0, dst_sb0, src_sb1, dst_sb1,
             asg0, asg1, adg0, adg1, hg0, hg1, exb, msg0, msg1,
             sem_as0, sem_as1, sem_ad0, sem_ad1, sem_hg0, sem_hg1,
             sem_sct0, sem_sct1, sem_lsa, sem_lsb):
    cid = lax.axis_index("c")
    sid = lax.axis_index("s")

    for m in (msg0, msg1):
        def _zrow(i, _, m=m):
            for off in list(range(0, W_ACC - 15, 16)) + [W_ACC - 16]:
                m[i, pl.ds(off, 16)] = jnp.zeros((16,), jnp.float32)
            return 0
        lax.fori_loop(0, CHUNK, _zrow, 0)
    def _zcopy(k, _):
        pltpu.sync_copy(msg0, acc.at[pl.ds(sid * ROWS_PT + k * CHUNK, CHUNK)])
        return 0
    lax.fori_loop(0, ROWS_PT // CHUNK, _zcopy, 0)
    plsc.subcore_barrier()

    asg = (asg0, asg1)
    adg = (adg0, adg1)
    hg = (hg0, hg1)
    msg = (msg0, msg1)
    sem_as = (sem_as0, sem_as1)
    sem_ad = (sem_ad0, sem_ad1)
    sem_hg = (sem_hg0, sem_hg1)
    sem_sct = (sem_sct0, sem_sct1)

    def _metapath(src_h, dst_h, as_h, ad_h, hsrc_h):
        row0 = sid * NCHUNKS
        sbs = ((src_sb0, dst_sb0), (src_sb1, dst_sb1))

        def _load_super(s, j):
            pltpu.sync_copy(src_h.at[pl.ds(row0 + s * SUPER, SUPER)], sbs[j][0])
            pltpu.sync_copy(dst_h.at[pl.ds(row0 + s * SUPER, SUPER)], sbs[j][1])

        def _issue(j, k, b):
            pltpu.async_copy(as_h.at[sbs[j][0].at[k]], asg[b], sem_as[b])
            pltpu.async_copy(ad_h.at[sbs[j][1].at[k]], adg[b], sem_ad[b])
            pltpu.async_copy(hsrc_h.at[sbs[j][0].at[k]], hg[b], sem_hg[b])

        def _wait_sct(b):
            pltpu.make_async_copy(msg[b], acc.at[dst_sb0.at[0]],
                                  sem_sct[b]).wait()

        def _chunk(s, half, k):
            b = k % 2
            if k == SUPER - 4:
                @pl.when(s < NSUPER - 1)
                def _():
                    pltpu.async_copy(
                        src_h.at[pl.ds(row0 + (s + 1) * SUPER, SUPER)],
                        sbs[1 - half][0], sem_lsa)
                    pltpu.async_copy(
                        dst_h.at[pl.ds(row0 + (s + 1) * SUPER, SUPER)],
                        sbs[1 - half][1], sem_lsb)
            if k < SUPER - 1:
                _issue(half, k + 1, 1 - b)
            else:
                @pl.when(s < NSUPER - 1)
                def _():
                    pltpu.make_async_copy(src_h.at[pl.ds(row0, SUPER)],
                                          sbs[1 - half][0], sem_lsa).wait()
                    pltpu.make_async_copy(dst_h.at[pl.ds(row0, SUPER)],
                                          sbs[1 - half][1], sem_lsb).wait()
                    _issue(1 - half, 0, 1 - b)
            _wait_sct(b)
            pltpu.make_async_copy(as_h.at[src_sb0.at[0]], asg[b], sem_as[b]).wait()
            pltpu.make_async_copy(ad_h.at[dst_sb0.at[0]], adg[b], sem_ad[b]).wait()

            @plsc.parallel_loop(0, CHUNK, step=1, unroll=2)
            def _edge(i):
                a = asg[b][i, :] + adg[b][i, :]
                a = jnp.where(a > 0, a, a * 0.2)
                exb[i, :] = jnp.exp(a)

            pltpu.make_async_copy(hsrc_h.at[src_sb0.at[0]], hg[b], sem_hg[b]).wait()

            @plsc.parallel_loop(0, CHUNK, step=1, unroll=2)
            def _edge2(i):
                ex = exb[i, :]
                msg[b][i, pl.ds(0, 16)] = ex
                for j in range(4):
                    raw = hg[b][i, pl.ds(32 * j, 32)]
                    va, vb = plsc.unpack(raw, format=plsc.PackFormat.INTERLEAVED)
                    msg[b][i, pl.ds(8 + 32 * j, 16)] = (
                        _lane_bcast(ex, 2 * j) * va)
                    msg[b][i, pl.ds(24 + 32 * j, 16)] = (
                        _lane_bcast(ex, 2 * j + 1) * vb)

            pltpu.async_copy(msg[b], acc.at[sbs[half][1].at[k]], sem_sct[b],
                             add=True)

        _load_super(0, 0)
        _load_super(1, 1)
        _issue(0, 0, 0)
        pltpu.async_copy(msg0, acc.at[dst_sb0.at[SUPER - 2]], sem_sct0, add=True)
        pltpu.async_copy(msg1, acc.at[dst_sb0.at[SUPER - 1]], sem_sct1, add=True)

        def _spair(sp, _):
            for half in range(2):
                s = 2 * sp + half
                for k in range(SUPER):
                    _chunk(s, half, k)
            return 0
        lax.fori_loop(0, NSUPER // 2, _spair, 0)
        _wait_sct(0)
        _wait_sct(1)

    @pl.when(cid == 0)
    def _():
        _metapath(src_uo, dst_uo, tsuo, tduo, hub)

    @pl.when(cid == 1)
    def _():
        _metapath(src_oo, dst_oo, tsoo, tdoo, hob)

    plsc.subcore_barrier()
    pltpu.sync_copy(acc.at[pl.ds(sid * ROWS_PT, ROWS_PT)],
                    out_hbm.at[cid, pl.ds(sid * ROWS_PT, ROWS_PT)])


def _stage2(ei_uo, ei_oo, tsuo, tduo, hub, tsoo, tdoo, hob):
    i32 = jnp.int32
    pad = EPAD - E

    def _prep(ei):
        src = jnp.concatenate([ei[0], jnp.zeros((pad,), i32)])
        dst = jnp.concatenate([ei[1], jnp.full((pad,), N, i32)])
        return src.reshape(EPAD // CHUNK, CHUNK), dst.reshape(EPAD // CHUNK, CHUNK)

    src_uo, dst_uo = _prep(ei_uo)
    src_oo, dst_oo = _prep(ei_oo)
    tduo_p = jnp.pad(tduo, ((0, 8), (0, 0)))
    tdoo_p = jnp.pad(tdoo, ((0, 8), (0, 0)))
    f32 = jnp.float32
    k = pl.kernel(
        _sc_body,
        out_type=jax.ShapeDtypeStruct((2, NPAD, W_ACC), f32),
        mesh=plsc.VectorSubcoreMesh(core_axis_name="c", subcore_axis_name="s"),
        compiler_params=pltpu.CompilerParams(use_tc_tiling_on_sc=False,
                                             needs_layout_passes=False),
        scratch_types=[
            pltpu.VMEM_SHARED((NPAD, W_ACC), f32),
            pltpu.VMEM((SUPER, CHUNK), i32),
            pltpu.VMEM((SUPER, CHUNK), i32),
            pltpu.VMEM((SUPER, CHUNK), i32),
            pltpu.VMEM((SUPER, CHUNK), i32),
            pltpu.VMEM((CHUNK, 16), f32),
            pltpu.VMEM((CHUNK, 16), f32),
            pltpu.VMEM((CHUNK, 16), f32),
            pltpu.VMEM((CHUNK, 16), f32),
            pltpu.VMEM((CHUNK, DH), jnp.bfloat16),
            pltpu.VMEM((CHUNK, DH), jnp.bfloat16),
            pltpu.VMEM((CHUNK, 16), f32),
            pltpu.VMEM((CHUNK, W_ACC), f32),
            pltpu.VMEM((CHUNK, W_ACC), f32),
            pltpu.SemaphoreType.DMA,
            pltpu.SemaphoreType.DMA,
            pltpu.SemaphoreType.DMA,
            pltpu.SemaphoreType.DMA,
            pltpu.SemaphoreType.DMA,
            pltpu.SemaphoreType.DMA,
            pltpu.SemaphoreType.DMA,
            pltpu.SemaphoreType.DMA,
            pltpu.SemaphoreType.DMA,
            pltpu.SemaphoreType.DMA,
        ],
    )
    return k(src_uo, dst_uo, src_oo, dst_oo, tsuo, tduo_p, hub, tsoo, tdoo_p, hob)


def _k3a_body(t_ref, r_ref, kwt_ref, kb_ref, o_ref, s_ref):
    pid = pl.program_id(0)
    t = t_ref[...]
    accum = t[:, :, 8:W_ACC]
    den = t[:, :, 0:8].reshape(2 * BN, 8)
    den128 = jnp.dot(den, r_ref[...],
                     preferred_element_type=jnp.float32).reshape(2, BN, DH)
    o = jnp.maximum(accum / (den128 + 1e-16), 0.0)
    o_ref[...] = o
    kk = jnp.tanh(jnp.dot(o.reshape(2 * BN, DH), kwt_ref[...],
                          preferred_element_type=jnp.float32) + kb_ref[...])
    part = kk.reshape(2, BN, DH).sum(axis=1)

    @pl.when(pid == 0)
    def _():
        s_ref[...] = jnp.zeros_like(s_ref)
    s_ref[...] += part


def _k3b_body(o_ref, s_ref, q_ref, lwt_ref, lb_ref, out_ref):
    s = s_ref[...]
    qv = q_ref[...]
    sc0 = jnp.sum(qv[0] * s[0]) / N
    sc1 = jnp.sum(qv[0] * s[1]) / N
    m = jnp.maximum(sc0, sc1)
    e0 = jnp.exp(sc0 - m)
    e1 = jnp.exp(sc1 - m)
    tot = e0 + e1
    o = o_ref[...]
    comb = (e0 / tot) * o[0] + (e1 / tot) * o[1]
    out_ref[...] = jnp.dot(comb, lwt_ref[...],
                           preferred_element_type=jnp.float32) + lb_ref[...]


def _stage3(sc_out, k_lin_W, k_lin_b, q, lin_W, lin_b):
    f32 = jnp.float32
    fixed2 = lambda i: (0, 0)
    r = jnp.repeat(jnp.eye(H, dtype=f32), D, axis=1)
    o, s = pl.pallas_call(
        _k3a_body,
        grid=(GRID,),
        in_specs=[
            pl.BlockSpec((2, BN, W_ACC), lambda i: (0, i, 0)),
            pl.BlockSpec((8, DH), fixed2),
            pl.BlockSpec((DH, DH), fixed2),
            pl.BlockSpec((1, DH), fixed2),
        ],
        out_specs=[
            pl.BlockSpec((2, BN, DH), lambda i: (0, i, 0)),
            pl.BlockSpec((2, DH), fixed2),
        ],
        out_shape=[
            jax.ShapeDtypeStruct((2, N, DH), f32),
            jax.ShapeDtypeStruct((2, DH), f32),
        ],
    )(sc_out, r, k_lin_W.T, k_lin_b.reshape(1, DH))
    return pl.pallas_call(
        _k3b_body,
        grid=(GRID,),
        in_specs=[
            pl.BlockSpec((2, BN, DH), lambda i: (0, i, 0)),
            pl.BlockSpec((2, DH), fixed2),
            pl.BlockSpec((1, DH), fixed2),
            pl.BlockSpec((DH, DOUT), fixed2),
            pl.BlockSpec((1, DOUT), fixed2),
        ],
        out_specs=pl.BlockSpec((BN, DOUT), lambda i: (i, 0)),
        out_shape=jax.ShapeDtypeStruct((N, DOUT), f32),
    )(o, s, q, lin_W.T, lin_b.reshape(1, DOUT))


def kernel(x_order, x_user, edge_index_user__to__order,
           edge_index_order__self__order, W_order, b_order, W_user, b_user,
           att_src_uo, att_dst_uo, att_src_oo, att_dst_oo,
           k_lin_W, k_lin_b, q, lin_W, lin_b):
    hob, hub, tsuo, tduo, tsoo, tdoo = _stage1(
        x_order, x_user, W_order, b_order, W_user, b_user,
        att_src_uo, att_dst_uo, att_src_oo, att_dst_oo)
    sc_out = _stage2(edge_index_user__to__order, edge_index_order__self__order,
                     tsuo, tduo, hub, tsoo, tdoo, hob)
    return _stage3(sc_out[:, :N, :], k_lin_W, k_lin_b, q, lin_W, lin_b)


# speedup vs baseline: 1.0059x; 1.0059x over previous
import jax
import jax.numpy as jnp
import numpy as np
from jax import lax
from jax.experimental import pallas as pl
from jax.experimental.pallas import tpu as pltpu
from jax.experimental.pallas import tpu_sc as plsc

H, D = 8, 16
N = 10000
E = 320000
DIN, DH, DOUT = 128, 128, 64

NTILES = 16
CHUNK = 64
SUPER = 8
EPT = 20480
EPAD = EPT * NTILES
NCHUNKS = EPT // CHUNK
NSUPER = NCHUNKS // SUPER
ROWS_PT = 640
NPAD = ROWS_PT * NTILES
W_ACC = 136
BN = 1000
GRID = N // BN


def _k1_body(xo_ref, xu_ref, wot_ref, wut_ref, bo_ref, bu_ref, p_ref,
             a_suo_ref, a_duo_ref, a_soo_ref, a_doo_ref,
             hob_ref, hub_ref, tsuo_ref, tduo_ref, tsoo_ref, tdoo_ref):
    ho = jnp.dot(xo_ref[...], wot_ref[...],
                 preferred_element_type=jnp.float32) + bo_ref[...]
    hu = jnp.dot(xu_ref[...], wut_ref[...],
                 preferred_element_type=jnp.float32) + bu_ref[...]
    hob_ref[...] = jnp.dot(ho, p_ref[...],
                           preferred_element_type=jnp.float32).astype(jnp.bfloat16)
    hub_ref[...] = jnp.dot(hu, p_ref[...],
                           preferred_element_type=jnp.float32).astype(jnp.bfloat16)
    tsuo_ref[...] = jnp.dot(hu, a_suo_ref[...], preferred_element_type=jnp.float32)
    tduo_ref[...] = jnp.dot(ho, a_duo_ref[...], preferred_element_type=jnp.float32)
    tsoo_ref[...] = jnp.dot(ho, a_soo_ref[...], preferred_element_type=jnp.float32)
    tdoo_ref[...] = jnp.dot(ho, a_doo_ref[...], preferred_element_type=jnp.float32)


def _blockdiag(att):
    a = jnp.einsum('hd,hk->hdk', att[0], jnp.eye(H, dtype=att.dtype))
    return jnp.pad(a.reshape(H * D, H), ((0, 0), (0, 8)))


def _interleave_perm():
    p = np.zeros((DH, DH), np.float32)
    for j in range(4):
        for t in range(16):
            p[32 * j + t, 32 * j + 2 * t] = 1.0
            p[32 * j + 16 + t, 32 * j + 2 * t + 1] = 1.0
    return jnp.asarray(p)


def _stage1(x_order, x_user, W_order, b_order, W_user, b_user,
            att_src_uo, att_dst_uo, att_src_oo, att_dst_oo):
    row = lambda i: (i, 0)
    fixed = lambda i: (0, 0)
    f32 = jnp.float32
    return pl.pallas_call(
        _k1_body,
        grid=(GRID,),
        in_specs=[
            pl.BlockSpec((BN, DIN), row), pl.BlockSpec((BN, DIN), row),
            pl.BlockSpec((DIN, DH), fixed), pl.BlockSpec((DIN, DH), fixed),
            pl.BlockSpec((1, DH), fixed), pl.BlockSpec((1, DH), fixed),
            pl.BlockSpec((DH, DH), fixed),
            pl.BlockSpec((DH, 16), fixed), pl.BlockSpec((DH, 16), fixed),
            pl.BlockSpec((DH, 16), fixed), pl.BlockSpec((DH, 16), fixed),
        ],
        out_specs=[
            pl.BlockSpec((BN, DH), row), pl.BlockSpec((BN, DH), row),
            pl.BlockSpec((BN, 16), row), pl.BlockSpec((BN, 16), row),
            pl.BlockSpec((BN, 16), row), pl.BlockSpec((BN, 16), row),
        ],
        out_shape=[
            jax.ShapeDtypeStruct((N, DH), jnp.bfloat16),
            jax.ShapeDtypeStruct((N, DH), jnp.bfloat16),
            jax.ShapeDtypeStruct((N, 16), f32), jax.ShapeDtypeStruct((N, 16), f32),
            jax.ShapeDtypeStruct((N, 16), f32), jax.ShapeDtypeStruct((N, 16), f32),
        ],
    )(x_order, x_user, W_order.T, W_user.T,
      b_order.reshape(1, DH), b_user.reshape(1, DH), _interleave_perm(),
      _blockdiag(att_src_uo), _blockdiag(att_dst_uo),
      _blockdiag(att_src_oo), _blockdiag(att_dst_oo))


def _lane_bcast(x, h):
    idx = jnp.full((16, 1), h, jnp.int32)
    dn = lax.GatherDimensionNumbers(
        offset_dims=(), collapsed_slice_dims=(0,), start_index_map=(0,))
    return lax.gather(x, idx, dn, (1,),
                      mode=lax.GatherScatterMode.PROMISE_IN_BOUNDS)


def _sc_body(src_uo, dst_uo, src_oo, dst_oo,
             tsuo, tduo, hub, tsoo, tdoo, hob,
             out_hbm,
             acc, src_sb0, dst_sb0, src_sb1, dst_sb1,
             asg0, asg1, adg0, adg1, hg0, hg1, exb, msg0, msg1,
             sem_as0, sem_as1, sem_ad0, sem_ad1, sem_hg0, sem_hg1,
             sem_sct0, sem_sct1, sem_lsa, sem_lsb):
    cid = lax.axis_index("c")
    sid = lax.axis_index("s")

    for m in (msg0, msg1):
        def _zrow(i, _, m=m):
            for off in list(range(0, W_ACC - 15, 16)) + [W_ACC - 16]:
                m[i, pl.ds(off, 16)] = jnp.zeros((16,), jnp.float32)
            return 0
        lax.fori_loop(0, CHUNK, _zrow, 0)
    for k in range(ROWS_PT // CHUNK):
        pltpu.async_copy(msg0, acc.at[pl.ds(sid * ROWS_PT + k * CHUNK, CHUNK)],
                         sem_lsa)
    for k in range(ROWS_PT // CHUNK):
        pltpu.make_async_copy(msg0, acc.at[pl.ds(sid * ROWS_PT, CHUNK)],
                              sem_lsa).wait()
    plsc.subcore_barrier()

    asg = (asg0, asg1)
    adg = (adg0, adg1)
    hg = (hg0, hg1)
    msg = (msg0, msg1)
    sem_as = (sem_as0, sem_as1)
    sem_ad = (sem_ad0, sem_ad1)
    sem_hg = (sem_hg0, sem_hg1)
    sem_sct = (sem_sct0, sem_sct1)

    def _metapath(src_h, dst_h, as_h, ad_h, hsrc_h):
        row0 = sid * NCHUNKS
        sbs = ((src_sb0, dst_sb0), (src_sb1, dst_sb1))

        def _load_super(s, j):
            pltpu.sync_copy(src_h.at[pl.ds(row0 + s * SUPER, SUPER)], sbs[j][0])
            pltpu.sync_copy(dst_h.at[pl.ds(row0 + s * SUPER, SUPER)], sbs[j][1])

        def _issue(j, k, b):
            pltpu.async_copy(as_h.at[sbs[j][0].at[k]], asg[b], sem_as[b])
            pltpu.async_copy(ad_h.at[sbs[j][1].at[k]], adg[b], sem_ad[b])
            pltpu.async_copy(hsrc_h.at[sbs[j][0].at[k]], hg[b], sem_hg[b])

        def _wait_sct(b):
            pltpu.make_async_copy(msg[b], acc.at[dst_sb0.at[0]],
                                  sem_sct[b]).wait()

        def _chunk(s, half, k):
            b = k % 2
            if k == SUPER - 4:
                @pl.when(s < NSUPER - 1)
                def _():
                    pltpu.async_copy(
                        src_h.at[pl.ds(row0 + (s + 1) * SUPER, SUPER)],
                        sbs[1 - half][0], sem_lsa)
                    pltpu.async_copy(
                        dst_h.at[pl.ds(row0 + (s + 1) * SUPER, SUPER)],
                        sbs[1 - half][1], sem_lsb)
            if k < SUPER - 1:
                _issue(half, k + 1, 1 - b)
            else:
                @pl.when(s < NSUPER - 1)
                def _():
                    pltpu.make_async_copy(src_h.at[pl.ds(row0, SUPER)],
                                          sbs[1 - half][0], sem_lsa).wait()
                    pltpu.make_async_copy(dst_h.at[pl.ds(row0, SUPER)],
                                          sbs[1 - half][1], sem_lsb).wait()
                    _issue(1 - half, 0, 1 - b)
            _wait_sct(b)
            pltpu.make_async_copy(as_h.at[src_sb0.at[0]], asg[b], sem_as[b]).wait()
            pltpu.make_async_copy(ad_h.at[dst_sb0.at[0]], adg[b], sem_ad[b]).wait()

            @plsc.parallel_loop(0, CHUNK, step=1, unroll=2)
            def _edge(i):
                a = asg[b][i, :] + adg[b][i, :]
                a = jnp.where(a > 0, a, a * 0.2)
                exb[i, :] = jnp.exp(a)

            pltpu.make_async_copy(hsrc_h.at[src_sb0.at[0]], hg[b], sem_hg[b]).wait()

            @plsc.parallel_loop(0, CHUNK, step=1, unroll=2)
            def _edge2(i):
                ex = exb[i, :]
                msg[b][i, pl.ds(0, 16)] = ex
                for j in range(4):
                    raw = hg[b][i, pl.ds(32 * j, 32)]
                    va, vb = plsc.unpack(raw, format=plsc.PackFormat.INTERLEAVED)
                    msg[b][i, pl.ds(8 + 32 * j, 16)] = (
                        _lane_bcast(ex, 2 * j) * va)
                    msg[b][i, pl.ds(24 + 32 * j, 16)] = (
                        _lane_bcast(ex, 2 * j + 1) * vb)

            pltpu.async_copy(msg[b], acc.at[sbs[half][1].at[k]], sem_sct[b],
                             add=True)

        _load_super(0, 0)
        _load_super(1, 1)
        _issue(0, 0, 0)
        pltpu.async_copy(msg0, acc.at[dst_sb0.at[SUPER - 2]], sem_sct0, add=True)
        pltpu.async_copy(msg1, acc.at[dst_sb0.at[SUPER - 1]], sem_sct1, add=True)

        def _spair(sp, _):
            for half in range(2):
                s = 2 * sp + half
                for k in range(SUPER):
                    _chunk(s, half, k)
            return 0
        lax.fori_loop(0, NSUPER // 2, _spair, 0)
        _wait_sct(0)
        _wait_sct(1)

    @pl.when(cid == 0)
    def _():
        _metapath(src_uo, dst_uo, tsuo, tduo, hub)

    @pl.when(cid == 1)
    def _():
        _metapath(src_oo, dst_oo, tsoo, tdoo, hob)

    plsc.subcore_barrier()
    pltpu.sync_copy(acc.at[pl.ds(sid * ROWS_PT, ROWS_PT)],
                    out_hbm.at[cid, pl.ds(sid * ROWS_PT, ROWS_PT)])


def _stage2(ei_uo, ei_oo, tsuo, tduo, hub, tsoo, tdoo, hob):
    i32 = jnp.int32
    pad = EPAD - E

    def _prep(ei):
        src = jnp.concatenate([ei[0], jnp.zeros((pad,), i32)])
        dst = jnp.concatenate([ei[1], jnp.full((pad,), N, i32)])
        return src.reshape(EPAD // CHUNK, CHUNK), dst.reshape(EPAD // CHUNK, CHUNK)

    src_uo, dst_uo = _prep(ei_uo)
    src_oo, dst_oo = _prep(ei_oo)
    tduo_p = jnp.pad(tduo, ((0, 8), (0, 0)))
    tdoo_p = jnp.pad(tdoo, ((0, 8), (0, 0)))
    f32 = jnp.float32
    k = pl.kernel(
        _sc_body,
        out_type=jax.ShapeDtypeStruct((2, NPAD, W_ACC), f32),
        mesh=plsc.VectorSubcoreMesh(core_axis_name="c", subcore_axis_name="s"),
        compiler_params=pltpu.CompilerParams(use_tc_tiling_on_sc=False,
                                             needs_layout_passes=False),
        scratch_types=[
            pltpu.VMEM_SHARED((NPAD, W_ACC), f32),
            pltpu.VMEM((SUPER, CHUNK), i32),
            pltpu.VMEM((SUPER, CHUNK), i32),
            pltpu.VMEM((SUPER, CHUNK), i32),
            pltpu.VMEM((SUPER, CHUNK), i32),
            pltpu.VMEM((CHUNK, 16), f32),
            pltpu.VMEM((CHUNK, 16), f32),
            pltpu.VMEM((CHUNK, 16), f32),
            pltpu.VMEM((CHUNK, 16), f32),
            pltpu.VMEM((CHUNK, DH), jnp.bfloat16),
            pltpu.VMEM((CHUNK, DH), jnp.bfloat16),
            pltpu.VMEM((CHUNK, 16), f32),
            pltpu.VMEM((CHUNK, W_ACC), f32),
            pltpu.VMEM((CHUNK, W_ACC), f32),
            pltpu.SemaphoreType.DMA,
            pltpu.SemaphoreType.DMA,
            pltpu.SemaphoreType.DMA,
            pltpu.SemaphoreType.DMA,
            pltpu.SemaphoreType.DMA,
            pltpu.SemaphoreType.DMA,
            pltpu.SemaphoreType.DMA,
            pltpu.SemaphoreType.DMA,
            pltpu.SemaphoreType.DMA,
            pltpu.SemaphoreType.DMA,
        ],
    )
    return k(src_uo, dst_uo, src_oo, dst_oo, tsuo, tduo_p, hub, tsoo, tdoo_p, hob)


def _k3a_body(t_ref, r_ref, kwt_ref, kb_ref, lwt_ref, z_ref, s_ref):
    pid = pl.program_id(0)
    t = t_ref[...]
    accum = t[:, :, 8:W_ACC]
    den = t[:, :, 0:8].reshape(2 * BN, 8)
    den128 = jnp.dot(den, r_ref[...],
                     preferred_element_type=jnp.float32).reshape(2, BN, DH)
    o = jnp.maximum(accum / (den128 + 1e-16), 0.0).reshape(2 * BN, DH)
    z_ref[...] = jnp.dot(o, lwt_ref[...],
                         preferred_element_type=jnp.float32).reshape(2, BN, DOUT)
    kk = jnp.tanh(jnp.dot(o, kwt_ref[...],
                          preferred_element_type=jnp.float32) + kb_ref[...])
    part = kk.reshape(2, BN, DH).sum(axis=1)

    @pl.when(pid == 0)
    def _():
        s_ref[...] = jnp.zeros_like(s_ref)
    s_ref[...] += part


def _k3b_body(z_ref, s_ref, q_ref, lb_ref, out_ref):
    s = s_ref[...]
    qv = q_ref[...]
    sc0 = jnp.sum(qv[0] * s[0]) / N
    sc1 = jnp.sum(qv[0] * s[1]) / N
    m = jnp.maximum(sc0, sc1)
    e0 = jnp.exp(sc0 - m)
    e1 = jnp.exp(sc1 - m)
    tot = e0 + e1
    z = z_ref[...]
    out_ref[...] = (e0 / tot) * z[0] + (e1 / tot) * z[1] + lb_ref[...]


def _stage3(sc_out, k_lin_W, k_lin_b, q, lin_W, lin_b):
    f32 = jnp.float32
    fixed2 = lambda i: (0, 0)
    r = jnp.repeat(jnp.eye(H, dtype=f32), D, axis=1)
    z, s = pl.pallas_call(
        _k3a_body,
        grid=(GRID,),
        in_specs=[
            pl.BlockSpec((2, BN, W_ACC), lambda i: (0, i, 0)),
            pl.BlockSpec((8, DH), fixed2),
            pl.BlockSpec((DH, DH), fixed2),
            pl.BlockSpec((1, DH), fixed2),
            pl.BlockSpec((DH, DOUT), fixed2),
        ],
        out_specs=[
            pl.BlockSpec((2, BN, DOUT), lambda i: (0, i, 0)),
            pl.BlockSpec((2, DH), fixed2),
        ],
        out_shape=[
            jax.ShapeDtypeStruct((2, N, DOUT), f32),
            jax.ShapeDtypeStruct((2, DH), f32),
        ],
    )(sc_out, r, k_lin_W.T, k_lin_b.reshape(1, DH), lin_W.T)
    return pl.pallas_call(
        _k3b_body,
        grid=(GRID,),
        in_specs=[
            pl.BlockSpec((2, BN, DOUT), lambda i: (0, i, 0)),
            pl.BlockSpec((2, DH), fixed2),
            pl.BlockSpec((1, DH), fixed2),
            pl.BlockSpec((1, DOUT), fixed2),
        ],
        out_specs=pl.BlockSpec((BN, DOUT), lambda i: (i, 0)),
        out_shape=jax.ShapeDtypeStruct((N, DOUT), f32),
    )(z, s, q, lin_b.reshape(1, DOUT))


def kernel(x_order, x_user, edge_index_user__to__order,
           edge_index_order__self__order, W_order, b_order, W_user, b_user,
           att_src_uo, att_dst_uo, att_src_oo, att_dst_oo,
           k_lin_W, k_lin_b, q, lin_W, lin_b):
    hob, hub, tsuo, tduo, tsoo, tdoo = _stage1(
        x_order, x_user, W_order, b_order, W_user, b_user,
        att_src_uo, att_dst_uo, att_src_oo, att_dst_oo)
    sc_out = _stage2(edge_index_user__to__order, edge_index_order__self__order,
                     tsuo, tduo, hub, tsoo, tdoo, hob)
    return _stage3(sc_out[:, :N, :], k_lin_W, k_lin_b, q, lin_W, lin_b)


# speedup vs baseline: 1.0163x; 1.0104x over previous
import jax
import jax.numpy as jnp
import numpy as np
from jax import lax
from jax.experimental import pallas as pl
from jax.experimental.pallas import tpu as pltpu
from jax.experimental.pallas import tpu_sc as plsc

H, D = 8, 16
N = 10000
E = 320000
DIN, DH, DOUT = 128, 128, 64

NTILES = 16
CHUNK = 64
SUPER = 4
EPT = 20480
EPAD = EPT * NTILES
NCHUNKS = EPT // CHUNK
NSUPER = NCHUNKS // SUPER
ROWS_PT = 640
NPAD = ROWS_PT * NTILES
W_ACC = 136
BN = 1000
GRID = N // BN


def _k1_body(xo_ref, xu_ref, wot_ref, wut_ref, bo_ref, bu_ref, p_ref,
             a_suo_ref, a_duo_ref, a_soo_ref, a_doo_ref,
             hob_ref, hub_ref, tsuo_ref, tduo_ref, tsoo_ref, tdoo_ref):
    ho = jnp.dot(xo_ref[...], wot_ref[...],
                 preferred_element_type=jnp.float32) + bo_ref[...]
    hu = jnp.dot(xu_ref[...], wut_ref[...],
                 preferred_element_type=jnp.float32) + bu_ref[...]
    hob_ref[...] = jnp.dot(ho, p_ref[...],
                           preferred_element_type=jnp.float32).astype(jnp.bfloat16)
    hub_ref[...] = jnp.dot(hu, p_ref[...],
                           preferred_element_type=jnp.float32).astype(jnp.bfloat16)
    tsuo_ref[...] = jnp.dot(hu, a_suo_ref[...], preferred_element_type=jnp.float32)
    tduo_ref[...] = jnp.dot(ho, a_duo_ref[...], preferred_element_type=jnp.float32)
    tsoo_ref[...] = jnp.dot(ho, a_soo_ref[...], preferred_element_type=jnp.float32)
    tdoo_ref[...] = jnp.dot(ho, a_doo_ref[...], preferred_element_type=jnp.float32)


def _blockdiag(att):
    a = jnp.einsum('hd,hk->hdk', att[0], jnp.eye(H, dtype=att.dtype))
    return jnp.pad(a.reshape(H * D, H), ((0, 0), (0, 8)))


def _interleave_perm():
    p = np.zeros((DH, DH), np.float32)
    for j in range(4):
        for t in range(16):
            p[32 * j + t, 32 * j + 2 * t] = 1.0
            p[32 * j + 16 + t, 32 * j + 2 * t + 1] = 1.0
    return jnp.asarray(p)


def _stage1(x_order, x_user, W_order, b_order, W_user, b_user,
            att_src_uo, att_dst_uo, att_src_oo, att_dst_oo):
    row = lambda i: (i, 0)
    fixed = lambda i: (0, 0)
    f32 = jnp.float32
    return pl.pallas_call(
        _k1_body,
        grid=(GRID,),
        in_specs=[
            pl.BlockSpec((BN, DIN), row), pl.BlockSpec((BN, DIN), row),
            pl.BlockSpec((DIN, DH), fixed), pl.BlockSpec((DIN, DH), fixed),
            pl.BlockSpec((1, DH), fixed), pl.BlockSpec((1, DH), fixed),
            pl.BlockSpec((DH, DH), fixed),
            pl.BlockSpec((DH, 16), fixed), pl.BlockSpec((DH, 16), fixed),
            pl.BlockSpec((DH, 16), fixed), pl.BlockSpec((DH, 16), fixed),
        ],
        out_specs=[
            pl.BlockSpec((BN, DH), row), pl.BlockSpec((BN, DH), row),
            pl.BlockSpec((BN, 16), row), pl.BlockSpec((BN, 16), row),
            pl.BlockSpec((BN, 16), row), pl.BlockSpec((BN, 16), row),
        ],
        out_shape=[
            jax.ShapeDtypeStruct((N, DH), jnp.bfloat16),
            jax.ShapeDtypeStruct((N, DH), jnp.bfloat16),
            jax.ShapeDtypeStruct((N, 16), f32), jax.ShapeDtypeStruct((N, 16), f32),
            jax.ShapeDtypeStruct((N, 16), f32), jax.ShapeDtypeStruct((N, 16), f32),
        ],
    )(x_order, x_user, W_order.T, W_user.T,
      b_order.reshape(1, DH), b_user.reshape(1, DH), _interleave_perm(),
      _blockdiag(att_src_uo), _blockdiag(att_dst_uo),
      _blockdiag(att_src_oo), _blockdiag(att_dst_oo))


def _lane_bcast(x, h):
    idx = jnp.full((16, 1), h, jnp.int32)
    dn = lax.GatherDimensionNumbers(
        offset_dims=(), collapsed_slice_dims=(0,), start_index_map=(0,))
    return lax.gather(x, idx, dn, (1,),
                      mode=lax.GatherScatterMode.PROMISE_IN_BOUNDS)


def _sc_body(src_uo, dst_uo, src_oo, dst_oo,
             tsuo, tduo, hub, tsoo, tdoo, hob,
             out_hbm,
             acc, src_sb0, dst_sb0, src_sb1, dst_sb1,
             asg0, asg1, adg0, adg1, hg0, hg1, exb, msg0, msg1,
             sem_as0, sem_as1, sem_ad0, sem_ad1, sem_hg0, sem_hg1,
             sem_sct0, sem_sct1, sem_lsa, sem_lsb):
    cid = lax.axis_index("c")
    sid = lax.axis_index("s")

    for m in (msg0, msg1):
        def _zrow(i, _, m=m):
            for off in list(range(0, W_ACC - 15, 16)) + [W_ACC - 16]:
                m[i, pl.ds(off, 16)] = jnp.zeros((16,), jnp.float32)
            return 0
        lax.fori_loop(0, CHUNK, _zrow, 0)
    for k in range(ROWS_PT // CHUNK):
        pltpu.async_copy(msg0, acc.at[pl.ds(sid * ROWS_PT + k * CHUNK, CHUNK)],
                         sem_lsa)
    for k in range(ROWS_PT // CHUNK):
        pltpu.make_async_copy(msg0, acc.at[pl.ds(sid * ROWS_PT, CHUNK)],
                              sem_lsa).wait()
    plsc.subcore_barrier()

    asg = (asg0, asg1)
    adg = (adg0, adg1)
    hg = (hg0, hg1)
    msg = (msg0, msg1)
    sem_as = (sem_as0, sem_as1)
    sem_ad = (sem_ad0, sem_ad1)
    sem_hg = (sem_hg0, sem_hg1)
    sem_sct = (sem_sct0, sem_sct1)

    def _metapath(src_h, dst_h, as_h, ad_h, hsrc_h):
        row0 = sid * NCHUNKS
        sbs = ((src_sb0, dst_sb0), (src_sb1, dst_sb1))

        def _load_super(s, j):
            pltpu.sync_copy(src_h.at[pl.ds(row0 + s * SUPER, SUPER)], sbs[j][0])
            pltpu.sync_copy(dst_h.at[pl.ds(row0 + s * SUPER, SUPER)], sbs[j][1])

        def _issue(j, k, b):
            pltpu.async_copy(as_h.at[sbs[j][0].at[k]], asg[b], sem_as[b])
            pltpu.async_copy(ad_h.at[sbs[j][1].at[k]], adg[b], sem_ad[b])
            pltpu.async_copy(hsrc_h.at[sbs[j][0].at[k]], hg[b], sem_hg[b])

        def _wait_sct(b):
            pltpu.make_async_copy(msg[b], acc.at[dst_sb0.at[0]],
                                  sem_sct[b]).wait()

        def _chunk(s, half, k):
            b = k % 2
            if k == SUPER - 4:
                @pl.when(s < NSUPER - 1)
                def _():
                    pltpu.async_copy(
                        src_h.at[pl.ds(row0 + (s + 1) * SUPER, SUPER)],
                        sbs[1 - half][0], sem_lsa)
                    pltpu.async_copy(
                        dst_h.at[pl.ds(row0 + (s + 1) * SUPER, SUPER)],
                        sbs[1 - half][1], sem_lsb)
            if k < SUPER - 1:
                _issue(half, k + 1, 1 - b)
            else:
                @pl.when(s < NSUPER - 1)
                def _():
                    pltpu.make_async_copy(src_h.at[pl.ds(row0, SUPER)],
                                          sbs[1 - half][0], sem_lsa).wait()
                    pltpu.make_async_copy(dst_h.at[pl.ds(row0, SUPER)],
                                          sbs[1 - half][1], sem_lsb).wait()
                    _issue(1 - half, 0, 1 - b)
            _wait_sct(b)
            pltpu.make_async_copy(as_h.at[src_sb0.at[0]], asg[b], sem_as[b]).wait()
            pltpu.make_async_copy(ad_h.at[dst_sb0.at[0]], adg[b], sem_ad[b]).wait()

            @plsc.parallel_loop(0, CHUNK, step=1, unroll=2)
            def _edge(i):
                a = asg[b][i, :] + adg[b][i, :]
                a = jnp.where(a > 0, a, a * 0.2)
                exb[i, :] = jnp.exp(a)

            pltpu.make_async_copy(hsrc_h.at[src_sb0.at[0]], hg[b], sem_hg[b]).wait()

            @plsc.parallel_loop(0, CHUNK, step=1, unroll=3)
            def _edge2(i):
                ex = exb[i, :]
                msg[b][i, pl.ds(0, 16)] = ex
                for j in range(4):
                    raw = hg[b][i, pl.ds(32 * j, 32)]
                    va, vb = plsc.unpack(raw, format=plsc.PackFormat.INTERLEAVED)
                    msg[b][i, pl.ds(8 + 32 * j, 16)] = (
                        _lane_bcast(ex, 2 * j) * va)
                    msg[b][i, pl.ds(24 + 32 * j, 16)] = (
                        _lane_bcast(ex, 2 * j + 1) * vb)

            pltpu.async_copy(msg[b], acc.at[sbs[half][1].at[k]], sem_sct[b],
                             add=True)

        _load_super(0, 0)
        _load_super(1, 1)
        _issue(0, 0, 0)
        pltpu.async_copy(msg0, acc.at[dst_sb0.at[SUPER - 2]], sem_sct0, add=True)
        pltpu.async_copy(msg1, acc.at[dst_sb0.at[SUPER - 1]], sem_sct1, add=True)

        def _spair(sp, _):
            for half in range(2):
                s = 2 * sp + half
                for k in range(SUPER):
                    _chunk(s, half, k)
            return 0
        lax.fori_loop(0, NSUPER // 2, _spair, 0)
        _wait_sct(0)
        _wait_sct(1)

    @pl.when(cid == 0)
    def _():
        _metapath(src_uo, dst_uo, tsuo, tduo, hub)

    @pl.when(cid == 1)
    def _():
        _metapath(src_oo, dst_oo, tsoo, tdoo, hob)

    plsc.subcore_barrier()
    pltpu.sync_copy(acc.at[pl.ds(sid * ROWS_PT, ROWS_PT)],
                    out_hbm.at[cid, pl.ds(sid * ROWS_PT, ROWS_PT)])


def _stage2(ei_uo, ei_oo, tsuo, tduo, hub, tsoo, tdoo, hob):
    i32 = jnp.int32
    pad = EPAD - E

    def _prep(ei):
        src = jnp.concatenate([ei[0], jnp.zeros((pad,), i32)])
        dst = jnp.concatenate([ei[1], jnp.full((pad,), N, i32)])
        return src.reshape(EPAD // CHUNK, CHUNK), dst.reshape(EPAD // CHUNK, CHUNK)

    src_uo, dst_uo = _prep(ei_uo)
    src_oo, dst_oo = _prep(ei_oo)
    tduo_p = jnp.pad(tduo, ((0, 8), (0, 0)))
    tdoo_p = jnp.pad(tdoo, ((0, 8), (0, 0)))
    f32 = jnp.float32
    k = pl.kernel(
        _sc_body,
        out_type=jax.ShapeDtypeStruct((2, NPAD, W_ACC), f32),
        mesh=plsc.VectorSubcoreMesh(core_axis_name="c", subcore_axis_name="s"),
        compiler_params=pltpu.CompilerParams(use_tc_tiling_on_sc=False,
                                             needs_layout_passes=False),
        scratch_types=[
            pltpu.VMEM_SHARED((NPAD, W_ACC), f32),
            pltpu.VMEM((SUPER, CHUNK), i32),
            pltpu.VMEM((SUPER, CHUNK), i32),
            pltpu.VMEM((SUPER, CHUNK), i32),
            pltpu.VMEM((SUPER, CHUNK), i32),
            pltpu.VMEM((CHUNK, 16), f32),
            pltpu.VMEM((CHUNK, 16), f32),
            pltpu.VMEM((CHUNK, 16), f32),
            pltpu.VMEM((CHUNK, 16), f32),
            pltpu.VMEM((CHUNK, DH), jnp.bfloat16),
            pltpu.VMEM((CHUNK, DH), jnp.bfloat16),
            pltpu.VMEM((CHUNK, 16), f32),
            pltpu.VMEM((CHUNK, W_ACC), f32),
            pltpu.VMEM((CHUNK, W_ACC), f32),
            pltpu.SemaphoreType.DMA,
            pltpu.SemaphoreType.DMA,
            pltpu.SemaphoreType.DMA,
            pltpu.SemaphoreType.DMA,
            pltpu.SemaphoreType.DMA,
            pltpu.SemaphoreType.DMA,
            pltpu.SemaphoreType.DMA,
            pltpu.SemaphoreType.DMA,
            pltpu.SemaphoreType.DMA,
            pltpu.SemaphoreType.DMA,
        ],
    )
    return k(src_uo, dst_uo, src_oo, dst_oo, tsuo, tduo_p, hub, tsoo, tdoo_p, hob)


def _k3a_body(t_ref, r_ref, kwt_ref, kb_ref, lwt_ref, z_ref, s_ref):
    pid = pl.program_id(0)
    t = t_ref[...]
    accum = t[:, :, 8:W_ACC]
    den = t[:, :, 0:8].reshape(2 * BN, 8)
    den128 = jnp.dot(den, r_ref[...],
                     preferred_element_type=jnp.float32).reshape(2, BN, DH)
    o = jnp.maximum(accum / (den128 + 1e-16), 0.0).reshape(2 * BN, DH)
    z_ref[...] = jnp.dot(o, lwt_ref[...],
                         preferred_element_type=jnp.float32).reshape(2, BN, DOUT)
    kk = jnp.tanh(jnp.dot(o, kwt_ref[...],
                          preferred_element_type=jnp.float32) + kb_ref[...])
    part = kk.reshape(2, BN, DH).sum(axis=1)

    @pl.when(pid == 0)
    def _():
        s_ref[...] = jnp.zeros_like(s_ref)
    s_ref[...] += part


def _k3b_body(z_ref, s_ref, q_ref, lb_ref, out_ref):
    s = s_ref[...]
    qv = q_ref[...]
    sc0 = jnp.sum(qv[0] * s[0]) / N
    sc1 = jnp.sum(qv[0] * s[1]) / N
    m = jnp.maximum(sc0, sc1)
    e0 = jnp.exp(sc0 - m)
    e1 = jnp.exp(sc1 - m)
    tot = e0 + e1
    z = z_ref[...]
    out_ref[...] = (e0 / tot) * z[0] + (e1 / tot) * z[1] + lb_ref[...]


def _stage3(sc_out, k_lin_W, k_lin_b, q, lin_W, lin_b):
    f32 = jnp.float32
    fixed2 = lambda i: (0, 0)
    r = jnp.repeat(jnp.eye(H, dtype=f32), D, axis=1)
    z, s = pl.pallas_call(
        _k3a_body,
        grid=(GRID,),
        in_specs=[
            pl.BlockSpec((2, BN, W_ACC), lambda i: (0, i, 0)),
            pl.BlockSpec((8, DH), fixed2),
            pl.BlockSpec((DH, DH), fixed2),
            pl.BlockSpec((1, DH), fixed2),
            pl.BlockSpec((DH, DOUT), fixed2),
        ],
        out_specs=[
            pl.BlockSpec((2, BN, DOUT), lambda i: (0, i, 0)),
            pl.BlockSpec((2, DH), fixed2),
        ],
        out_shape=[
            jax.ShapeDtypeStruct((2, N, DOUT), f32),
            jax.ShapeDtypeStruct((2, DH), f32),
        ],
    )(sc_out, r, k_lin_W.T, k_lin_b.reshape(1, DH), lin_W.T)
    return pl.pallas_call(
        _k3b_body,
        grid=(GRID,),
        in_specs=[
            pl.BlockSpec((2, BN, DOUT), lambda i: (0, i, 0)),
            pl.BlockSpec((2, DH), fixed2),
            pl.BlockSpec((1, DH), fixed2),
            pl.BlockSpec((1, DOUT), fixed2),
        ],
        out_specs=pl.BlockSpec((BN, DOUT), lambda i: (i, 0)),
        out_shape=jax.ShapeDtypeStruct((N, DOUT), f32),
    )(z, s, q, lin_b.reshape(1, DOUT))


def kernel(x_order, x_user, edge_index_user__to__order,
           edge_index_order__self__order, W_order, b_order, W_user, b_user,
           att_src_uo, att_dst_uo, att_src_oo, att_dst_oo,
           k_lin_W, k_lin_b, q, lin_W, lin_b):
    hob, hub, tsuo, tduo, tsoo, tdoo = _stage1(
        x_order, x_user, W_order, b_order, W_user, b_user,
        att_src_uo, att_dst_uo, att_src_oo, att_dst_oo)
    sc_out = _stage2(edge_index_user__to__order, edge_index_order__self__order,
                     tsuo, tduo, hub, tsoo, tdoo, hob)
    return _stage3(sc_out[:, :N, :], k_lin_W, k_lin_b, q, lin_W, lin_b)
